# baseline (device time: 55497 ns/iter reference)
import jax
import jax.numpy as jnp
from jax import lax
from jax.experimental import pallas as pl
from jax.experimental.pallas import tpu as pltpu

M_BLOCK = 1024
EPS = 1e-6


def kernel(partial, gamma):
    _, m_total, d = partial.shape
    p2d = partial.reshape(m_total, d)
    g2d = gamma.reshape(1, d)

    def body(p_ref, g_ref, out_ref, x_recv, send_sem, recv_sem):
        my_x = lax.axis_index("x")
        my_y = lax.axis_index("y")
        other_x = 1 - my_x

        barrier_sem = pltpu.get_barrier_semaphore()
        pl.semaphore_signal(
            barrier_sem, inc=1,
            device_id=(other_x, my_y), device_id_type=pl.DeviceIdType.MESH,
        )
        pl.semaphore_wait(barrier_sem, 1)

        rdma = pltpu.make_async_remote_copy(
            src_ref=p_ref.at[pl.ds(other_x * M_BLOCK, M_BLOCK), :],
            dst_ref=x_recv,
            send_sem=send_sem,
            recv_sem=recv_sem,
            device_id=(other_x, my_y),
            device_id_type=pl.DeviceIdType.MESH,
        )
        rdma.start()
        rdma.wait()

        local = p_ref[pl.ds(my_x * M_BLOCK, M_BLOCK), :]
        s = local + x_recv[...]
        ms = jnp.mean(s * s, axis=-1, keepdims=True)
        out_ref[...] = s * lax.rsqrt(ms + EPS) * g_ref[...]

    return pl.pallas_call(
        body,
        out_shape=jax.ShapeDtypeStruct((M_BLOCK, d), jnp.float32),
        in_specs=[
            pl.BlockSpec(memory_space=pltpu.VMEM),
            pl.BlockSpec(memory_space=pltpu.VMEM),
        ],
        out_specs=pl.BlockSpec(memory_space=pltpu.VMEM),
        scratch_shapes=[
            pltpu.VMEM((M_BLOCK, d), jnp.float32),
            pltpu.SemaphoreType.DMA,
            pltpu.SemaphoreType.DMA,
        ],
        compiler_params=pltpu.CompilerParams(collective_id=0),
    )(p2d, g2d)


# device time: 37829 ns/iter; 1.4670x vs baseline; 1.4670x over previous
import jax
import jax.numpy as jnp
from jax import lax
from jax.experimental import pallas as pl
from jax.experimental.pallas import tpu as pltpu

M_BLOCK = 1024
HALF = 512
K = 8
C = HALF // K
EPS = 1e-6


def kernel(partial, gamma):
    _, m_total, d = partial.shape
    p2d = partial.reshape(m_total, d)
    g2d = gamma.reshape(1, d)

    def body(p_ref, g_ref, out_ref, x_recv, y_recv, s_buf,
             x_send_sems, x_recv_sems, y_send_sems, y_recv_sems):
        my_x = lax.axis_index("x")
        my_y = lax.axis_index("y")
        other_x = 1 - my_x
        other_y = 1 - my_y

        def norm(rows):
            ms = jnp.mean(rows * rows, axis=-1, keepdims=True)
            return rows * lax.rsqrt(ms + EPS) * g_ref[...]

        barrier_sem = pltpu.get_barrier_semaphore()
        for dev in ((other_x, my_y), (my_x, other_y)):
            pl.semaphore_signal(
                barrier_sem, inc=1,
                device_id=dev, device_id_type=pl.DeviceIdType.MESH,
            )
        pl.semaphore_wait(barrier_sem, 2)

        x_src_base = other_x * M_BLOCK + my_y * HALF
        x_rdmas = []
        for i in range(K):
            r = pltpu.make_async_remote_copy(
                src_ref=p_ref.at[pl.ds(x_src_base + i * C, C), :],
                dst_ref=x_recv.at[i],
                send_sem=x_send_sems.at[i],
                recv_sem=x_recv_sems.at[i],
                device_id=(other_x, my_y),
                device_id_type=pl.DeviceIdType.MESH,
            )
            r.start()
            x_rdmas.append(r)

        loc_base = my_x * M_BLOCK + my_y * HALF
        y_rdmas = []
        for i in range(K):
            x_rdmas[i].wait_recv()
            s = p_ref[pl.ds(loc_base + i * C, C), :] + x_recv[i]
            s_buf[i] = s
            r = pltpu.make_async_remote_copy(
                src_ref=s_buf.at[i],
                dst_ref=y_recv.at[i],
                send_sem=y_send_sems.at[i],
                recv_sem=y_recv_sems.at[i],
                device_id=(my_x, other_y),
                device_id_type=pl.DeviceIdType.MESH,
            )
            r.start()
            y_rdmas.append(r)
            out_ref[pl.ds(my_y * HALF + i * C, C), :] = norm(s)

        for i in range(K):
            y_rdmas[i].wait_recv()
            out_ref[pl.ds(other_y * HALF + i * C, C), :] = norm(y_recv[i])

        for i in range(K):
            x_rdmas[i].wait_send()
            y_rdmas[i].wait_send()

    return pl.pallas_call(
        body,
        out_shape=jax.ShapeDtypeStruct((M_BLOCK, d), jnp.float32),
        in_specs=[
            pl.BlockSpec(memory_space=pltpu.VMEM),
            pl.BlockSpec(memory_space=pltpu.VMEM),
        ],
        out_specs=pl.BlockSpec(memory_space=pltpu.VMEM),
        scratch_shapes=[
            pltpu.VMEM((K, C, d), jnp.float32),
            pltpu.VMEM((K, C, d), jnp.float32),
            pltpu.VMEM((K, C, d), jnp.float32),
            pltpu.SemaphoreType.DMA((K,)),
            pltpu.SemaphoreType.DMA((K,)),
            pltpu.SemaphoreType.DMA((K,)),
            pltpu.SemaphoreType.DMA((K,)),
        ],
        compiler_params=pltpu.CompilerParams(collective_id=0),
    )(p2d, g2d)


# device time: 37787 ns/iter; 1.4687x vs baseline; 1.0011x over previous
import jax
import jax.numpy as jnp
from jax import lax
from jax.experimental import pallas as pl
from jax.experimental.pallas import tpu as pltpu

M_BLOCK = 1024
HALF = 512
K = 8
C = HALF // K
EPS = 1e-6


def kernel(partial, gamma):
    _, m_total, d = partial.shape
    p2d = partial.reshape(m_total, d)
    g2d = gamma.reshape(1, d)

    def body(p_ref, g_ref, out_ref, x_recv,
             x_send_sems, x_recv_sems, y_send_sems, y_recv_sems):
        my_x = lax.axis_index("x")
        my_y = lax.axis_index("y")
        other_x = 1 - my_x
        other_y = 1 - my_y

        barrier_sem = pltpu.get_barrier_semaphore()
        for dev in ((other_x, my_y), (my_x, other_y)):
            pl.semaphore_signal(
                barrier_sem, inc=1,
                device_id=dev, device_id_type=pl.DeviceIdType.MESH,
            )
        pl.semaphore_wait(barrier_sem, 2)

        x_src_base = other_x * M_BLOCK + my_y * HALF
        x_rdmas = []
        for i in range(K):
            r = pltpu.make_async_remote_copy(
                src_ref=p_ref.at[pl.ds(x_src_base + i * C, C), :],
                dst_ref=x_recv.at[i],
                send_sem=x_send_sems.at[i],
                recv_sem=x_recv_sems.at[i],
                device_id=(other_x, my_y),
                device_id_type=pl.DeviceIdType.MESH,
            )
            r.start()
            x_rdmas.append(r)

        loc_base = my_x * M_BLOCK + my_y * HALF
        y_rdmas = []
        for i in range(K):
            x_rdmas[i].wait_recv()
            s = p_ref[pl.ds(loc_base + i * C, C), :] + x_recv[i]
            ms = jnp.mean(s * s, axis=-1, keepdims=True)
            out_slice = pl.ds(my_y * HALF + i * C, C)
            out_ref[out_slice, :] = s * lax.rsqrt(ms + EPS) * g_ref[...]
            r = pltpu.make_async_remote_copy(
                src_ref=out_ref.at[out_slice, :],
                dst_ref=out_ref.at[out_slice, :],
                send_sem=y_send_sems.at[i],
                recv_sem=y_recv_sems.at[i],
                device_id=(my_x, other_y),
                device_id_type=pl.DeviceIdType.MESH,
            )
            r.start()
            y_rdmas.append(r)

        for i in range(K):
            y_rdmas[i].wait_recv()
        for i in range(K):
            x_rdmas[i].wait_send()
            y_rdmas[i].wait_send()

    return pl.pallas_call(
        body,
        out_shape=jax.ShapeDtypeStruct((M_BLOCK, d), jnp.float32),
        in_specs=[
            pl.BlockSpec(memory_space=pltpu.VMEM),
            pl.BlockSpec(memory_space=pltpu.VMEM),
        ],
        out_specs=pl.BlockSpec(memory_space=pltpu.VMEM),
        scratch_shapes=[
            pltpu.VMEM((K, C, d), jnp.float32),
            pltpu.SemaphoreType.DMA((K,)),
            pltpu.SemaphoreType.DMA((K,)),
            pltpu.SemaphoreType.DMA((K,)),
            pltpu.SemaphoreType.DMA((K,)),
        ],
        compiler_params=pltpu.CompilerParams(collective_id=0),
    )(p2d, g2d)


# device time: 36713 ns/iter; 1.5116x vs baseline; 1.0293x over previous
import jax
import jax.numpy as jnp
from jax import lax
from jax.experimental import pallas as pl
from jax.experimental.pallas import tpu as pltpu

M_BLOCK = 1024
HALF = 512
import os
K = int(os.environ.get("RSRMS_K", "8"))
C = HALF // K
EPS = 1e-6


def kernel(partial, gamma):
    _, m_total, d = partial.shape
    p2d = partial.reshape(m_total, d)
    g2d = gamma.reshape(1, d)

    def body(p_ref, g_ref, out_ref, x_recv,
             x_send_sems, x_recv_sems, y_send_sems, y_recv_sems):
        my_x = lax.axis_index("x")
        my_y = lax.axis_index("y")
        other_x = 1 - my_x
        other_y = 1 - my_y

        barrier_sem = pltpu.get_barrier_semaphore()
        for dev in ((other_x, my_y), (my_x, other_y)):
            pl.semaphore_signal(
                barrier_sem, inc=1,
                device_id=dev, device_id_type=pl.DeviceIdType.MESH,
            )
        pl.semaphore_wait(barrier_sem, 2)

        x_src_base = other_x * M_BLOCK + my_y * HALF
        x_rdmas = []
        for i in range(K):
            r = pltpu.make_async_remote_copy(
                src_ref=p_ref.at[pl.ds(x_src_base + i * C, C), :],
                dst_ref=x_recv.at[i],
                send_sem=x_send_sems.at[i],
                recv_sem=x_recv_sems.at[i],
                device_id=(other_x, my_y),
                device_id_type=pl.DeviceIdType.MESH,
            )
            r.start()
            x_rdmas.append(r)

        loc_base = my_x * M_BLOCK + my_y * HALF
        y_rdmas = []
        for i in range(K):
            x_rdmas[i].wait_recv()
            s = p_ref[pl.ds(loc_base + i * C, C), :] + x_recv[i]
            ms = jnp.mean(s * s, axis=-1, keepdims=True)
            out_slice = pl.ds(my_y * HALF + i * C, C)
            out_ref[out_slice, :] = s * lax.rsqrt(ms + EPS) * g_ref[...]
            r = pltpu.make_async_remote_copy(
                src_ref=out_ref.at[out_slice, :],
                dst_ref=out_ref.at[out_slice, :],
                send_sem=y_send_sems.at[i],
                recv_sem=y_recv_sems.at[i],
                device_id=(my_x, other_y),
                device_id_type=pl.DeviceIdType.MESH,
            )
            r.start()
            y_rdmas.append(r)

        for i in range(K):
            y_rdmas[i].wait_recv()
        for i in range(K):
            x_rdmas[i].wait_send()
            y_rdmas[i].wait_send()

    return pl.pallas_call(
        body,
        out_shape=jax.ShapeDtypeStruct((M_BLOCK, d), jnp.float32),
        in_specs=[
            pl.BlockSpec(memory_space=pltpu.VMEM),
            pl.BlockSpec(memory_space=pltpu.VMEM),
        ],
        out_specs=pl.BlockSpec(memory_space=pltpu.VMEM),
        scratch_shapes=[
            pltpu.VMEM((K, C, d), jnp.float32),
            pltpu.SemaphoreType.DMA((K,)),
            pltpu.SemaphoreType.DMA((K,)),
            pltpu.SemaphoreType.DMA((K,)),
            pltpu.SemaphoreType.DMA((K,)),
        ],
        compiler_params=pltpu.CompilerParams(collective_id=0),
    )(p2d, g2d)
